# baseline (speedup 1.0000x reference)
import functools

import jax
import jax.numpy as jnp
from jax import lax
from jax.experimental import pallas as pl
from jax.experimental.pallas import tpu as pltpu
from jax.experimental.pallas import tpu_sc as plsc

_P = 20
_C, _H, _W = 3, 4000, 4000
_LK = _H // _P
_L = _LK * _LK
_M = _L // 2
_NT = 16
_IDX_PAD = 20480
_CHUNKS = _IDX_PAD // (_NT * 128)
_GW = 16
_RPT = _L // _NT


def _mask_body(idx_hbm, mask_hbm, buf_v, idx_v, z_v, sem):
    cid = lax.axis_index("c")
    sid = lax.axis_index("s")

    @pl.when(cid == 0)
    def _():
        def fill_ones(i, _):
            buf_v[i] = jnp.ones((_GW,), jnp.float32)
            return 0

        lax.fori_loop(0, _RPT, fill_ones, 0)

        def fill_zeros(i, _):
            z_v[i] = jnp.zeros((_GW,), jnp.float32)
            return 0

        lax.fori_loop(0, 128, fill_zeros, 0)

        pltpu.sync_copy(buf_v, mask_hbm.at[pl.ds(sid * _RPT, _RPT)])
        pltpu.sync_copy(idx_hbm.at[pl.ds(sid * _CHUNKS, _CHUNKS)], idx_v)
        plsc.subcore_barrier()
        copies = [
            pltpu.async_copy(z_v, mask_hbm.at[idx_v.at[j]], sem)
            for j in range(_CHUNKS)
        ]
        for c in copies:
            c.wait()


@functools.cache
def _get_build_mask():
    return functools.partial(
        pl.kernel,
        out_type=jax.ShapeDtypeStruct((_L, _GW), jnp.float32),
        mesh=plsc.VectorSubcoreMesh(core_axis_name="c", subcore_axis_name="s"),
        scratch_types=[
            pltpu.VMEM((_RPT, _GW), jnp.float32),
            pltpu.VMEM((_CHUNKS, 128), jnp.int32),
            pltpu.VMEM((128, _GW), jnp.float32),
            pltpu.SemaphoreType.DMA,
        ],
        compiler_params=pltpu.CompilerParams(use_tc_tiling_on_sc=False),
    )(_mask_body)


def _expand_body(m_ref, out_ref):
    i1 = lax.broadcasted_iota(jnp.int32, (_LK * _GW, _LK), 0)
    c1 = lax.broadcasted_iota(jnp.int32, (_LK * _GW, _LK), 1) * _GW
    sel = (i1 == c1).astype(jnp.float32)
    mc = jnp.dot(m_ref[...], sel, preferred_element_type=jnp.float32)
    i2 = lax.broadcasted_iota(jnp.int32, (_LK, _W), 0)
    c2 = lax.broadcasted_iota(jnp.int32, (_LK, _W), 1) // _P
    rep = (i2 == c2).astype(jnp.float32)
    out_ref[...] = jnp.dot(mc, rep, preferred_element_type=jnp.float32)


def _expand_mask(mask_gw):
    return pl.pallas_call(
        _expand_body,
        out_shape=jax.ShapeDtypeStruct((_LK, _W), jnp.float32),
    )(mask_gw)


_RB = 400


def _mul_body(img_ref, mask_ref, out_ref):
    rows = lax.broadcasted_iota(jnp.int32, (_RB, _RB // _P), 0) // _P
    cols = lax.broadcasted_iota(jnp.int32, (_RB, _RB // _P), 1)
    oneh = (rows == cols).astype(jnp.float32)
    mexp = jnp.dot(oneh, mask_ref[0], preferred_element_type=jnp.float32)
    out_ref[0] = img_ref[0] * mexp


def _apply_mask(image, mask3d):
    grid = (_C, _H // _RB)
    return pl.pallas_call(
        _mul_body,
        grid=grid,
        in_specs=[
            pl.BlockSpec((1, _RB, _W), lambda c, r: (c, r, 0)),
            pl.BlockSpec((1, _RB // _P, _W), lambda c, r: (r, 0, 0)),
        ],
        out_specs=pl.BlockSpec((1, _RB, _W), lambda c, r: (c, r, 0)),
        out_shape=jax.ShapeDtypeStruct((_C, _H, _W), jnp.float32),
        compiler_params=pltpu.CompilerParams(
            dimension_semantics=("parallel", "parallel"),
        ),
    )(image, mask3d)


@jax.jit
def kernel(image, rand_idx):
    pad = jnp.broadcast_to(rand_idx[:1], (_IDX_PAD - _M,))
    idx2d = jnp.concatenate([rand_idx, pad]).reshape(_IDX_PAD // 128, 128)
    mask = _get_build_mask()(idx2d)
    maskw = _expand_mask(mask.reshape(_LK, _LK * _GW))
    return _apply_mask(image, maskw.reshape(_H // _RB, _RB // _P, _W))


# speedup vs baseline: 1.0368x; 1.0368x over previous
import functools

import jax
import jax.numpy as jnp
from jax import lax
from jax.experimental import pallas as pl
from jax.experimental.pallas import tpu as pltpu
from jax.experimental.pallas import tpu_sc as plsc

_P = 20
_C, _H, _W = 3, 4000, 4000
_LK = _H // _P
_L = _LK * _LK
_M = _L // 2
_NT = 16
_IDX_PAD = 20480
_CHUNKS = _IDX_PAD // (_NT * 128)
_GW = 16
_RPT = _L // _NT


def _mask_body(idx_hbm, mask_hbm, buf_v, idx_v, z_v, sem):
    cid = lax.axis_index("c")
    sid = lax.axis_index("s")

    @pl.when(cid == 0)
    def _():
        def fill_ones(i, _):
            buf_v[i] = jnp.ones((_GW,), jnp.float32)
            return 0

        lax.fori_loop(0, _RPT, fill_ones, 0)

        def fill_zeros(i, _):
            z_v[i] = jnp.zeros((_GW,), jnp.float32)
            return 0

        lax.fori_loop(0, 128, fill_zeros, 0)

        pltpu.sync_copy(buf_v, mask_hbm.at[pl.ds(sid * _RPT, _RPT)])
        pltpu.sync_copy(idx_hbm.at[pl.ds(sid * _CHUNKS, _CHUNKS)], idx_v)
        plsc.subcore_barrier()
        copies = [
            pltpu.async_copy(z_v, mask_hbm.at[idx_v.at[j]], sem)
            for j in range(_CHUNKS)
        ]
        for c in copies:
            c.wait()


@functools.cache
def _get_build_mask():
    return functools.partial(
        pl.kernel,
        out_type=jax.ShapeDtypeStruct((_L, _GW), jnp.float32),
        mesh=plsc.VectorSubcoreMesh(core_axis_name="c", subcore_axis_name="s"),
        scratch_types=[
            pltpu.VMEM((_RPT, _GW), jnp.float32),
            pltpu.VMEM((_CHUNKS, 128), jnp.int32),
            pltpu.VMEM((128, _GW), jnp.float32),
            pltpu.SemaphoreType.DMA,
        ],
        compiler_params=pltpu.CompilerParams(use_tc_tiling_on_sc=False),
    )(_mask_body)


def _expand_body(m_ref, out_ref):
    i1 = lax.broadcasted_iota(jnp.int32, (_LK * _GW, _LK), 0)
    c1 = lax.broadcasted_iota(jnp.int32, (_LK * _GW, _LK), 1) * _GW
    sel = (i1 == c1).astype(jnp.float32)
    mc = jnp.dot(m_ref[...], sel, preferred_element_type=jnp.float32)
    i2 = lax.broadcasted_iota(jnp.int32, (_LK, _W), 0)
    c2 = lax.broadcasted_iota(jnp.int32, (_LK, _W), 1) // _P
    rep = (i2 == c2).astype(jnp.float32)
    out_ref[...] = jnp.dot(mc, rep, preferred_element_type=jnp.float32)


def _expand_mask(mask_gw):
    return pl.pallas_call(
        _expand_body,
        out_shape=jax.ShapeDtypeStruct((_LK, _W), jnp.float32),
    )(mask_gw)


_RB = 800


def _mul_body(img_ref, mask_ref, out_ref):
    rows = lax.broadcasted_iota(jnp.int32, (_RB, _RB // _P), 0) // _P
    cols = lax.broadcasted_iota(jnp.int32, (_RB, _RB // _P), 1)
    oneh = (rows == cols).astype(jnp.float32)
    mexp = jnp.dot(oneh, mask_ref[0], preferred_element_type=jnp.float32)
    out_ref[0] = img_ref[0] * mexp


def _apply_mask(image, mask3d):
    grid = (_C, _H // _RB)
    return pl.pallas_call(
        _mul_body,
        grid=grid,
        in_specs=[
            pl.BlockSpec((1, _RB, _W), lambda c, r: (c, r, 0)),
            pl.BlockSpec((1, _RB // _P, _W), lambda c, r: (r, 0, 0)),
        ],
        out_specs=pl.BlockSpec((1, _RB, _W), lambda c, r: (c, r, 0)),
        out_shape=jax.ShapeDtypeStruct((_C, _H, _W), jnp.float32),
        compiler_params=pltpu.CompilerParams(
            dimension_semantics=("parallel", "parallel"),
            vmem_limit_bytes=120 * 1024 * 1024,
        ),
    )(image, mask3d)


@jax.jit
def kernel(image, rand_idx):
    pad = jnp.broadcast_to(rand_idx[:1], (_IDX_PAD - _M,))
    idx2d = jnp.concatenate([rand_idx, pad]).reshape(_IDX_PAD // 128, 128)
    mask = _get_build_mask()(idx2d)
    maskw = _expand_mask(mask.reshape(_LK, _LK * _GW))
    return _apply_mask(image, maskw.reshape(_H // _RB, _RB // _P, _W))


# speedup vs baseline: 1.1068x; 1.0675x over previous
import functools

import jax
import jax.numpy as jnp
from jax import lax
from jax.experimental import pallas as pl
from jax.experimental.pallas import tpu as pltpu
from jax.experimental.pallas import tpu_sc as plsc

_P = 20
_C, _H, _W = 3, 4000, 4000
_LK = _H // _P
_L = _LK * _LK
_M = _L // 2
_NT = 16
_IDX_PAD = 20480
_CHUNKS = _IDX_PAD // (_NT * 128)
_GW = 16
_RPT = _L // _NT
_FILL = 125


def _mask_body(idx_hbm, mask_hbm, buf_v, idx_v, z_v, sem):
    cid = lax.axis_index("c")
    sid = lax.axis_index("s")

    @pl.when(cid == 0)
    def _():
        for i in range(_FILL):
            buf_v[i] = jnp.ones((_GW,), jnp.float32)
        for i in range(128):
            z_v[i] = jnp.zeros((_GW,), jnp.float32)

        init = [
            pltpu.async_copy(
                buf_v, mask_hbm.at[pl.ds(sid * _RPT + k * _FILL, _FILL)], sem
            )
            for k in range(_RPT // _FILL)
        ]
        pltpu.sync_copy(idx_hbm.at[pl.ds(sid * _CHUNKS, _CHUNKS)], idx_v)
        for c in init:
            c.wait()
        plsc.subcore_barrier()
        copies = [
            pltpu.async_copy(z_v, mask_hbm.at[idx_v.at[j]], sem)
            for j in range(_CHUNKS)
        ]
        for c in copies:
            c.wait()


@functools.cache
def _get_build_mask():
    return functools.partial(
        pl.kernel,
        out_type=jax.ShapeDtypeStruct((_L, _GW), jnp.float32),
        mesh=plsc.VectorSubcoreMesh(core_axis_name="c", subcore_axis_name="s"),
        scratch_types=[
            pltpu.VMEM((_FILL, _GW), jnp.float32),
            pltpu.VMEM((_CHUNKS, 128), jnp.int32),
            pltpu.VMEM((128, _GW), jnp.float32),
            pltpu.SemaphoreType.DMA,
        ],
        compiler_params=pltpu.CompilerParams(use_tc_tiling_on_sc=False),
    )(_mask_body)


def _expand_body(m_ref, out_ref):
    i1 = lax.broadcasted_iota(jnp.int32, (_LK * _GW, _LK), 0)
    c1 = lax.broadcasted_iota(jnp.int32, (_LK * _GW, _LK), 1) * _GW
    sel = (i1 == c1).astype(jnp.float32)
    mc = jnp.dot(m_ref[...], sel, preferred_element_type=jnp.float32)
    i2 = lax.broadcasted_iota(jnp.int32, (_LK, _W), 0)
    c2 = lax.broadcasted_iota(jnp.int32, (_LK, _W), 1) // _P
    rep = (i2 == c2).astype(jnp.float32)
    out_ref[...] = jnp.dot(mc, rep, preferred_element_type=jnp.float32)


def _expand_mask(mask_gw):
    return pl.pallas_call(
        _expand_body,
        out_shape=jax.ShapeDtypeStruct((_LK, _W), jnp.float32),
    )(mask_gw)


_RB = 800


def _mul_body(img_ref, mask_ref, out_ref):
    rows = lax.broadcasted_iota(jnp.int32, (_RB, _RB // _P), 0) // _P
    cols = lax.broadcasted_iota(jnp.int32, (_RB, _RB // _P), 1)
    oneh = (rows == cols).astype(jnp.float32)
    mexp = jnp.dot(oneh, mask_ref[0], preferred_element_type=jnp.float32)
    out_ref[0] = img_ref[0] * mexp


def _apply_mask(image, mask3d):
    grid = (_C, _H // _RB)
    return pl.pallas_call(
        _mul_body,
        grid=grid,
        in_specs=[
            pl.BlockSpec((1, _RB, _W), lambda c, r: (c, r, 0)),
            pl.BlockSpec((1, _RB // _P, _W), lambda c, r: (r, 0, 0)),
        ],
        out_specs=pl.BlockSpec((1, _RB, _W), lambda c, r: (c, r, 0)),
        out_shape=jax.ShapeDtypeStruct((_C, _H, _W), jnp.float32),
        compiler_params=pltpu.CompilerParams(
            dimension_semantics=("parallel", "parallel"),
            vmem_limit_bytes=120 * 1024 * 1024,
        ),
    )(image, mask3d)


@jax.jit
def kernel(image, rand_idx):
    pad = jnp.broadcast_to(rand_idx[:1], (_IDX_PAD - _M,))
    idx2d = jnp.concatenate([rand_idx, pad]).reshape(_IDX_PAD // 128, 128)
    mask = _get_build_mask()(idx2d)
    maskw = _expand_mask(mask.reshape(_LK, _LK * _GW))
    return _apply_mask(image, maskw.reshape(_H // _RB, _RB // _P, _W))


# speedup vs baseline: 1.1356x; 1.0261x over previous
import functools

import jax
import jax.numpy as jnp
from jax import lax
from jax.experimental import pallas as pl
from jax.experimental.pallas import tpu as pltpu
from jax.experimental.pallas import tpu_sc as plsc

_P = 20
_C, _H, _W = 3, 4000, 4000
_LK = _H // _P
_L = _LK * _LK
_M = _L // 2
_NT = 16
_IDX_PAD = 20480
_CHUNKS = _IDX_PAD // (_NT * 128)
_GW = 16
_RPT = _L // _NT
_FILL = 125


def _mask_body(idx_hbm, mask_hbm, buf_v, idx_v, z_v, sem):
    cid = lax.axis_index("c")
    sid = lax.axis_index("s")

    @pl.when(cid == 0)
    def _():
        for i in range(_FILL):
            buf_v[i] = jnp.ones((_GW,), jnp.float32)
        for i in range(128):
            z_v[i] = jnp.zeros((_GW,), jnp.float32)

        init = [
            pltpu.async_copy(
                buf_v, mask_hbm.at[pl.ds(sid * _RPT + k * _FILL, _FILL)], sem
            )
            for k in range(_RPT // _FILL)
        ]
        pltpu.sync_copy(idx_hbm.at[pl.ds(sid * _CHUNKS, _CHUNKS)], idx_v)
        for c in init:
            c.wait()
        plsc.subcore_barrier()
        copies = [
            pltpu.async_copy(z_v, mask_hbm.at[idx_v.at[j]], sem)
            for j in range(_CHUNKS)
        ]
        for c in copies:
            c.wait()


@functools.cache
def _get_build_mask():
    return functools.partial(
        pl.kernel,
        out_type=jax.ShapeDtypeStruct((_L, _GW), jnp.float32),
        mesh=plsc.VectorSubcoreMesh(core_axis_name="c", subcore_axis_name="s"),
        scratch_types=[
            pltpu.VMEM((_FILL, _GW), jnp.float32),
            pltpu.VMEM((_CHUNKS, 128), jnp.int32),
            pltpu.VMEM((128, _GW), jnp.float32),
            pltpu.SemaphoreType.DMA,
        ],
        compiler_params=pltpu.CompilerParams(use_tc_tiling_on_sc=False),
    )(_mask_body)


_RB = 800
_PR = _RB // _P


def _mul_body(img_ref, mask_ref, out_ref, sel_ref, rep_ref):
    c = pl.program_id(0)
    r = pl.program_id(1)

    @pl.when(jnp.logical_and(c == 0, r == 0))
    def _():
        i1 = lax.broadcasted_iota(jnp.int32, (_LK * _GW, _LK), 0)
        c1 = lax.broadcasted_iota(jnp.int32, (_LK * _GW, _LK), 1) * _GW
        sel_ref[...] = (i1 == c1).astype(jnp.float32)
        i2 = lax.broadcasted_iota(jnp.int32, (_LK, _W), 0)
        c2 = lax.broadcasted_iota(jnp.int32, (_LK, _W), 1) // _P
        rep_ref[...] = (i2 == c2).astype(jnp.float32)

    mc = jnp.dot(mask_ref[0], sel_ref[...],
                 preferred_element_type=jnp.float32)
    mw = jnp.dot(mc, rep_ref[...],
                 preferred_element_type=jnp.float32)
    rows = lax.broadcasted_iota(jnp.int32, (_RB, _PR), 0) // _P
    cols = lax.broadcasted_iota(jnp.int32, (_RB, _PR), 1)
    oneh = (rows == cols).astype(jnp.float32)
    mexp = jnp.dot(oneh, mw, preferred_element_type=jnp.float32)
    out_ref[0] = img_ref[0] * mexp


def _apply_mask(image, mask3d):
    grid = (_C, _H // _RB)
    return pl.pallas_call(
        _mul_body,
        grid=grid,
        in_specs=[
            pl.BlockSpec((1, _RB, _W), lambda c, r: (c, r, 0)),
            pl.BlockSpec((1, _PR, _LK * _GW), lambda c, r: (r, 0, 0)),
        ],
        out_specs=pl.BlockSpec((1, _RB, _W), lambda c, r: (c, r, 0)),
        out_shape=jax.ShapeDtypeStruct((_C, _H, _W), jnp.float32),
        scratch_shapes=[
            pltpu.VMEM((_LK * _GW, _LK), jnp.float32),
            pltpu.VMEM((_LK, _W), jnp.float32),
        ],
        compiler_params=pltpu.CompilerParams(
            dimension_semantics=("arbitrary", "arbitrary"),
            vmem_limit_bytes=120 * 1024 * 1024,
        ),
    )(image, mask3d)


@jax.jit
def kernel(image, rand_idx):
    pad = jnp.broadcast_to(rand_idx[:1], (_IDX_PAD - _M,))
    idx2d = jnp.concatenate([rand_idx, pad]).reshape(_IDX_PAD // 128, 128)
    mask = _get_build_mask()(idx2d)
    return _apply_mask(image, mask.reshape(_H // _RB, _PR, _LK * _GW))


# speedup vs baseline: 1.1455x; 1.0087x over previous
import functools

import jax
import jax.numpy as jnp
from jax import lax
from jax.experimental import pallas as pl
from jax.experimental.pallas import tpu as pltpu
from jax.experimental.pallas import tpu_sc as plsc

_P = 20
_C, _H, _W = 3, 4000, 4000
_LK = _H // _P
_L = _LK * _LK
_M = _L // 2
_NT = 16
_IDX_PAD = 20480
_CHUNKS = _IDX_PAD // (_NT * 128)
_GW = 16
_RPT = _L // _NT
_FILL = 125


def _mask_body(idx_hbm, mask_hbm, buf_v, idx_v, z_v, sem):
    cid = lax.axis_index("c")
    sid = lax.axis_index("s")

    @pl.when(cid == 0)
    def _():
        for i in range(_FILL):
            buf_v[i] = jnp.ones((_GW,), jnp.float32)
        for i in range(128):
            z_v[i] = jnp.zeros((_GW,), jnp.float32)

        init = [
            pltpu.async_copy(
                buf_v, mask_hbm.at[pl.ds(sid * _RPT + k * _FILL, _FILL)], sem
            )
            for k in range(_RPT // _FILL)
        ]
        pltpu.sync_copy(idx_hbm.at[pl.ds(sid * _CHUNKS, _CHUNKS)], idx_v)
        for c in init:
            c.wait()
        plsc.subcore_barrier()
        copies = [
            pltpu.async_copy(z_v, mask_hbm.at[idx_v.at[j]], sem)
            for j in range(_CHUNKS)
        ]
        for c in copies:
            c.wait()


@functools.cache
def _get_build_mask():
    return functools.partial(
        pl.kernel,
        out_type=jax.ShapeDtypeStruct((_L, _GW), jnp.float32),
        mesh=plsc.VectorSubcoreMesh(core_axis_name="c", subcore_axis_name="s"),
        scratch_types=[
            pltpu.VMEM((_FILL, _GW), jnp.float32),
            pltpu.VMEM((_CHUNKS, 128), jnp.int32),
            pltpu.VMEM((128, _GW), jnp.float32),
            pltpu.SemaphoreType.DMA,
        ],
        compiler_params=pltpu.CompilerParams(use_tc_tiling_on_sc=False),
    )(_mask_body)


_RB = 800
_PR = _RB // _P


def _mul_body(img_ref, mask_ref, out_ref, sel_ref, rep_ref, oneh_ref):
    r = pl.program_id(0)
    c = pl.program_id(1)

    @pl.when(jnp.logical_and(c == 0, r == 0))
    def _():
        i1 = lax.broadcasted_iota(jnp.int32, (_LK * _GW, _LK), 0)
        c1 = lax.broadcasted_iota(jnp.int32, (_LK * _GW, _LK), 1) * _GW
        sel_ref[...] = (i1 == c1).astype(jnp.float32)
        i2 = lax.broadcasted_iota(jnp.int32, (_LK, _W), 0)
        c2 = lax.broadcasted_iota(jnp.int32, (_LK, _W), 1) // _P
        rep_ref[...] = (i2 == c2).astype(jnp.float32)
        i3 = lax.broadcasted_iota(jnp.int32, (_RB, _PR), 0) // _P
        c3 = lax.broadcasted_iota(jnp.int32, (_RB, _PR), 1)
        oneh_ref[...] = (i3 == c3).astype(jnp.float32)

    mc = jnp.dot(mask_ref[0], sel_ref[...],
                 preferred_element_type=jnp.float32)
    mw = jnp.dot(mc, rep_ref[...],
                 preferred_element_type=jnp.float32)
    mexp = jnp.dot(oneh_ref[...], mw,
                   preferred_element_type=jnp.float32)
    out_ref[0] = img_ref[0] * mexp


def _apply_mask(image, mask3d):
    grid = (_H // _RB, _C)
    return pl.pallas_call(
        _mul_body,
        grid=grid,
        in_specs=[
            pl.BlockSpec((1, _RB, _W), lambda r, c: (c, r, 0)),
            pl.BlockSpec((1, _PR, _LK * _GW), lambda r, c: (r, 0, 0)),
        ],
        out_specs=pl.BlockSpec((1, _RB, _W), lambda r, c: (c, r, 0)),
        out_shape=jax.ShapeDtypeStruct((_C, _H, _W), jnp.float32),
        scratch_shapes=[
            pltpu.VMEM((_LK * _GW, _LK), jnp.float32),
            pltpu.VMEM((_LK, _W), jnp.float32),
            pltpu.VMEM((_RB, _PR), jnp.float32),
        ],
        compiler_params=pltpu.CompilerParams(
            dimension_semantics=("arbitrary", "arbitrary"),
            vmem_limit_bytes=120 * 1024 * 1024,
        ),
    )(image, mask3d)


@jax.jit
def kernel(image, rand_idx):
    pad = jnp.broadcast_to(rand_idx[:1], (_IDX_PAD - _M,))
    idx2d = jnp.concatenate([rand_idx, pad]).reshape(_IDX_PAD // 128, 128)
    mask = _get_build_mask()(idx2d)
    return _apply_mask(image, mask.reshape(_H // _RB, _PR, _LK * _GW))
